# baseline (speedup 1.0000x reference)
import jax
import jax.numpy as jnp
from jax.experimental import pallas as pl

_B = 10000
_STRIDE = 3000
_NIDS = 64


def _copy_kernel(x_ref, o_ref):
    i = pl.program_id(0)
    j = pl.program_id(1)
    o_ref[0] = x_ref[0]
    for k in range(_NIDS):
        rid = k * _STRIDE

        @pl.when((i == 0) & (j == rid // _B))
        def _zero_row(rid=rid):
            o_ref[0, rid % _B, :] = jnp.zeros((128,), jnp.float32)


def kernel(data):
    n = data.shape[1]
    return pl.pallas_call(
        _copy_kernel,
        grid=(data.shape[0], n // _B),
        in_specs=[pl.BlockSpec((1, _B, 128), lambda i, j: (i, j, 0))],
        out_specs=pl.BlockSpec((1, _B, 128), lambda i, j: (i, j, 0)),
        out_shape=jax.ShapeDtypeStruct(data.shape, data.dtype),
    )(data)


# speedup vs baseline: 1.0097x; 1.0097x over previous
import jax
import jax.numpy as jnp
from jax.experimental import pallas as pl

_B = 25000
_STRIDE = 3000
_NIDS = 64


def _copy_kernel(x_ref, o_ref):
    i = pl.program_id(0)
    j = pl.program_id(1)
    o_ref[0] = x_ref[0]
    for k in range(_NIDS):
        rid = k * _STRIDE

        @pl.when((i == 0) & (j == rid // _B))
        def _zero_row(rid=rid):
            o_ref[0, rid % _B, :] = jnp.zeros((128,), jnp.float32)


def kernel(data):
    n = data.shape[1]
    return pl.pallas_call(
        _copy_kernel,
        grid=(data.shape[0], n // _B),
        in_specs=[pl.BlockSpec((1, _B, 128), lambda i, j: (i, j, 0))],
        out_specs=pl.BlockSpec((1, _B, 128), lambda i, j: (i, j, 0)),
        out_shape=jax.ShapeDtypeStruct(data.shape, data.dtype),
    )(data)
